# Initial kernel scaffold; baseline (speedup 1.0000x reference)
import jax
import jax.numpy as jnp
from jax.experimental import pallas as pl


def kernel(feat_map, W_node, b_node, ln_g, ln_b, W_pos1, b_pos1, W_pos2, b_pos2, W_edge1, b_edge1, W_edge2, b_edge2):
    raise NotImplementedError("write your pallas kernel here")


# speedup vs baseline: 6.1617x; 6.1617x over previous
import functools

import jax
import jax.numpy as jnp
from jax import lax
from jax.experimental import pallas as pl
from jax.experimental.pallas import tpu as pltpu
from jax.experimental.pallas import tpu_sc as plsc

_K = 8
_SQRT_HALF = 0.7071067811865476


def _gelu(x):
    return 0.5 * x * (1.0 + lax.erf(x * jnp.float32(_SQRT_HALF)))
_NC = 2
_NS = 16
_NW = _NC * _NS
_CH = 128


def _main_body(tok_ref, Wn_ref, bn_ref, g_ref, bln_ref, Wa_ref, Wb_ref, be1_ref,
               nodes_ref, adj_ref, gidx_ref, a_ref, b_ref):
    N = tok_ref.shape[1]
    tok = tok_ref[0]
    h = jnp.dot(tok, Wn_ref[...], preferred_element_type=jnp.float32) + bn_ref[...]
    mu = jnp.mean(h, axis=-1, keepdims=True)
    var = jnp.mean((h - mu) ** 2, axis=-1, keepdims=True)
    h = (h - mu) / jnp.sqrt(var + 1e-5) * g_ref[...] + bln_ref[...]
    nodes = _gelu(h)
    nodes_ref[0] = nodes
    ss = jnp.sum(nodes * nodes, axis=-1, keepdims=True)
    norm = jnp.maximum(jnp.sqrt(ss), 1e-12)
    nrm = nodes / norm
    sim = lax.dot_general(nrm, nrm, (((1,), (1,)), ((), ())),
                          preferred_element_type=jnp.float32)
    rid = lax.broadcasted_iota(jnp.int32, (N, N), 0)
    cid = lax.broadcasted_iota(jnp.int32, (N, N), 1)
    sim = jnp.where(rid == cid, sim - 1e9, sim)
    cols = []
    s = sim
    for _ in range(_K):
        vmax = jnp.max(s, axis=-1, keepdims=True)
        idx = jnp.min(jnp.where(s == vmax, cid, N), axis=-1, keepdims=True)
        cols.append(idx)
        s = jnp.where(cid == idx, -jnp.inf, s)
    adj = jnp.concatenate(cols, axis=-1)
    adj_ref[0] = adj
    gidx_ref[0] = adj + pl.program_id(0) * N
    a_ref[0] = jnp.dot(nodes, Wa_ref[...], preferred_element_type=jnp.float32) + be1_ref[...]
    b_ref[0] = jnp.dot(nodes, Wb_ref[...], preferred_element_type=jnp.float32)


def _gather_body(n_chunks, table_hbm, idx_hbm, out_hbm, idx_v, rows_v, sem):
    wid = lax.axis_index("s") * _NC + lax.axis_index("c")

    def body(j, carry):
        base = wid * (n_chunks * _CH) + j * _CH
        pltpu.sync_copy(idx_hbm.at[pl.ds(base, _CH)], idx_v)
        pltpu.async_copy(table_hbm.at[idx_v], rows_v, sem).wait()
        pltpu.sync_copy(rows_v, out_hbm.at[pl.ds(base, _CH)])
        return carry

    lax.fori_loop(0, n_chunks, body, 0)


def _edge_body(h_grid, a_ref, nb_ref, adj_ref, Wp1_ref, bp1_ref, Wp2_ref, bp2_ref,
               Wc_ref, We2_ref, be2_ref, out_ref):
    N = a_ref.shape[1]
    Wc_comb = jnp.dot(Wp2_ref[...], Wc_ref[...], preferred_element_type=jnp.float32)
    bc_comb = jnp.dot(bp2_ref[...], Wc_ref[...], preferred_element_type=jnp.float32)
    a = a_ref[0]
    step = jnp.float32(1.0 / (h_grid - 1))
    n_iota = lax.broadcasted_iota(jnp.int32, (N, 1), 0)
    yi = n_iota // h_grid
    xi = n_iota % h_grid
    w0 = Wp1_ref[0:1, :]
    w1 = Wp1_ref[1:2, :]
    for k in range(_K):
        adj_k = adj_ref[0, :, k, :]
        yj = adj_k // h_grid
        xj = adj_k % h_grid
        dy = (yj - yi).astype(jnp.float32) * step
        dx = (xj - xi).astype(jnp.float32) * step
        g1 = _gelu(dy * w0 + dx * w1 + bp1_ref[...])
        ck = jnp.dot(g1, Wc_comb, preferred_element_type=jnp.float32) + bc_comb
        hidden = _gelu(a + nb_ref[0, :, k, :] + ck)
        out_ref[0, :, k, :] = (
            jnp.dot(hidden, We2_ref[...], preferred_element_type=jnp.float32)
            + be2_ref[...])


def kernel(feat_map, W_node, b_node, ln_g, ln_b, W_pos1, b_pos1, W_pos2, b_pos2,
           W_edge1, b_edge1, W_edge2, b_edge2):
    B, C, H, W = feat_map.shape
    N = H * W
    D = W_node.shape[1]
    K = _K

    tokens = feat_map.reshape(B, C, N).transpose(0, 2, 1)
    Wa = W_edge1[:D]
    Wb = W_edge1[D:2 * D]
    Wc = W_edge1[2 * D:]
    bn2 = b_node.reshape(1, D)
    g2 = ln_g.reshape(1, D)
    bln2 = ln_b.reshape(1, D)
    be1_2 = b_edge1.reshape(1, D)
    bp1_2 = b_pos1.reshape(1, 64)
    bp2_2 = b_pos2.reshape(1, D)
    be2_2 = b_edge2.reshape(1, D)

    f32 = jnp.float32
    nodes, adj, gidx, a_c, b_c = pl.pallas_call(
        _main_body,
        grid=(B,),
        in_specs=[
            pl.BlockSpec((1, N, C), lambda i: (i, 0, 0)),
            pl.BlockSpec((C, D), lambda i: (0, 0)),
            pl.BlockSpec((1, D), lambda i: (0, 0)),
            pl.BlockSpec((1, D), lambda i: (0, 0)),
            pl.BlockSpec((1, D), lambda i: (0, 0)),
            pl.BlockSpec((D, D), lambda i: (0, 0)),
            pl.BlockSpec((D, D), lambda i: (0, 0)),
            pl.BlockSpec((1, D), lambda i: (0, 0)),
        ],
        out_specs=[
            pl.BlockSpec((1, N, D), lambda i: (i, 0, 0)),
            pl.BlockSpec((1, N, K), lambda i: (i, 0, 0)),
            pl.BlockSpec((1, N, K), lambda i: (i, 0, 0)),
            pl.BlockSpec((1, N, D), lambda i: (i, 0, 0)),
            pl.BlockSpec((1, N, D), lambda i: (i, 0, 0)),
        ],
        out_shape=[
            jax.ShapeDtypeStruct((B, N, D), f32),
            jax.ShapeDtypeStruct((B, N, K), jnp.int32),
            jax.ShapeDtypeStruct((B, N, K), jnp.int32),
            jax.ShapeDtypeStruct((B, N, D), f32),
            jax.ShapeDtypeStruct((B, N, D), f32),
        ],
    )(tokens, W_node, bn2, g2, bln2, Wa, Wb, be1_2)

    total = B * N * K
    n_chunks = total // (_NW * _CH)
    mesh = plsc.VectorSubcoreMesh(core_axis_name="c", subcore_axis_name="s")
    nb = pl.kernel(
        functools.partial(_gather_body, n_chunks),
        mesh=mesh,
        out_type=jax.ShapeDtypeStruct((total, D), f32),
        scratch_types=[
            pltpu.VMEM((_CH,), jnp.int32),
            pltpu.VMEM((_CH, D), f32),
            pltpu.SemaphoreType.DMA,
        ],
    )(b_c.reshape(B * N, D), gidx.reshape(total))

    edges = pl.pallas_call(
        functools.partial(_edge_body, H),
        grid=(B,),
        in_specs=[
            pl.BlockSpec((1, N, D), lambda i: (i, 0, 0)),
            pl.BlockSpec((1, N, K, D), lambda i: (i, 0, 0, 0)),
            pl.BlockSpec((1, N, K, 1), lambda i: (i, 0, 0, 0)),
            pl.BlockSpec((2, 64), lambda i: (0, 0)),
            pl.BlockSpec((1, 64), lambda i: (0, 0)),
            pl.BlockSpec((64, D), lambda i: (0, 0)),
            pl.BlockSpec((1, D), lambda i: (0, 0)),
            pl.BlockSpec((D, D), lambda i: (0, 0)),
            pl.BlockSpec((D, D), lambda i: (0, 0)),
            pl.BlockSpec((1, D), lambda i: (0, 0)),
        ],
        out_specs=pl.BlockSpec((1, N, K, D), lambda i: (i, 0, 0, 0)),
        out_shape=jax.ShapeDtypeStruct((B, N, K, D), f32),
    )(a_c, nb.reshape(B, N, K, D), adj.reshape(B, N, K, 1),
      W_pos1, bp1_2, W_pos2, bp2_2, Wc, W_edge2, be2_2)

    return (nodes, edges, adj)


# speedup vs baseline: 11.1720x; 1.8132x over previous
import functools

import jax
import jax.numpy as jnp
from jax import lax
from jax.experimental import pallas as pl
from jax.experimental.pallas import tpu as pltpu
from jax.experimental.pallas import tpu_sc as plsc

_K = 8
_SQRT_HALF = 0.7071067811865476
_NC = 2
_NS = 16
_NW = _NC * _NS
_CH = 128


def _gelu(x):
    return 0.5 * x * (1.0 + lax.erf(x * jnp.float32(_SQRT_HALF)))


def _ctable_body(h_grid, Wp1_ref, bp1_ref, Wp2_ref, bp2_ref, Wc_ref, out_ref):
    T = out_ref.shape[0]
    S = 2 * h_grid - 1
    step = jnp.float32(1.0 / (h_grid - 1))
    Wc_comb = jnp.dot(Wp2_ref[...], Wc_ref[...], preferred_element_type=jnp.float32)
    bc_comb = jnp.dot(bp2_ref[...], Wc_ref[...], preferred_element_type=jnp.float32)
    d = lax.broadcasted_iota(jnp.int32, (T, 1), 0)
    dy = (d // S - (h_grid - 1)).astype(jnp.float32) * step
    dx = (d % S - (h_grid - 1)).astype(jnp.float32) * step
    g1 = _gelu(dy * Wp1_ref[0:1, :] + dx * Wp1_ref[1:2, :] + bp1_ref[...])
    out_ref[...] = jnp.dot(g1, Wc_comb, preferred_element_type=jnp.float32) + bc_comb


def _main_body(h_grid, fm_ref, Wn_ref, bn_ref, g_ref, bln_ref, Wa_ref, Wb_ref,
               be1_ref, nodes_ref, adj_ref, gidx_ref, cidx_ref, a_ref, b_ref):
    N = fm_ref.shape[2]
    fm = fm_ref[0]
    h = lax.dot_general(fm, Wn_ref[...], (((0,), (0,)), ((), ())),
                        preferred_element_type=jnp.float32) + bn_ref[...]
    mu = jnp.mean(h, axis=-1, keepdims=True)
    var = jnp.mean((h - mu) ** 2, axis=-1, keepdims=True)
    h = (h - mu) / jnp.sqrt(var + 1e-5) * g_ref[...] + bln_ref[...]
    nodes = _gelu(h)
    nodes_ref[0] = nodes
    ss = jnp.sum(nodes * nodes, axis=-1, keepdims=True)
    norm = jnp.maximum(jnp.sqrt(ss), 1e-12)
    nrm = nodes / norm
    sim = lax.dot_general(nrm, nrm, (((1,), (1,)), ((), ())),
                          preferred_element_type=jnp.float32)
    rid = lax.broadcasted_iota(jnp.int32, (N, N), 0)
    cid = lax.broadcasted_iota(jnp.int32, (N, N), 1)
    sim = jnp.where(rid == cid, sim - 1e9, sim)
    cols = []
    s = sim
    for _ in range(_K):
        vmax = jnp.max(s, axis=-1, keepdims=True)
        idx = jnp.min(jnp.where(s == vmax, cid, N), axis=-1, keepdims=True)
        cols.append(idx)
        s = jnp.where(cid == idx, -jnp.inf, s)
    adj = jnp.concatenate(cols, axis=-1)
    adj_ref[0] = adj
    gidx_ref[0] = adj + pl.program_id(0) * N
    S = 2 * h_grid - 1
    rown = lax.broadcasted_iota(jnp.int32, (N, _K), 0)
    dyi = adj // h_grid - rown // h_grid + (h_grid - 1)
    dxi = adj % h_grid - rown % h_grid + (h_grid - 1)
    cidx_ref[0] = dyi * S + dxi
    a_ref[0] = jnp.dot(nodes, Wa_ref[...], preferred_element_type=jnp.float32) + be1_ref[...]
    b_ref[0] = jnp.dot(nodes, Wb_ref[...], preferred_element_type=jnp.float32)


def _gather_body(n_chunks, btab_hbm, ctab_hbm, bidx_hbm, cidx_hbm,
                 nb_hbm, cc_hbm, bi_v, ci_v, brows_v, crows_v, bsem, csem):
    wid = lax.axis_index("s") * _NC + lax.axis_index("c")

    def body(j, carry):
        base = wid * (n_chunks * _CH) + j * _CH
        pltpu.sync_copy(bidx_hbm.at[pl.ds(base, _CH)], bi_v)
        pltpu.sync_copy(cidx_hbm.at[pl.ds(base, _CH)], ci_v)
        bcp = pltpu.async_copy(btab_hbm.at[bi_v], brows_v, bsem)
        ccp = pltpu.async_copy(ctab_hbm.at[ci_v], crows_v, csem)
        bcp.wait()
        ccp.wait()
        pltpu.sync_copy(brows_v, nb_hbm.at[pl.ds(base, _CH)])
        pltpu.sync_copy(crows_v, cc_hbm.at[pl.ds(base, _CH)])
        return carry

    lax.fori_loop(0, n_chunks, body, 0)


def _edge_body(a_ref, nb_ref, cc_ref, We2_ref, be2_ref, out_ref):
    a = a_ref[0]
    for k in range(_K):
        hidden = _gelu(a + nb_ref[0, :, k, :] + cc_ref[0, :, k, :])
        out_ref[0, :, k, :] = (
            jnp.dot(hidden, We2_ref[...], preferred_element_type=jnp.float32)
            + be2_ref[...])


def kernel(feat_map, W_node, b_node, ln_g, ln_b, W_pos1, b_pos1, W_pos2, b_pos2,
           W_edge1, b_edge1, W_edge2, b_edge2):
    B, C, H, W = feat_map.shape
    N = H * W
    D = W_node.shape[1]
    K = _K
    T = 2304

    fm3 = feat_map.reshape(B, C, N)
    Wa = W_edge1[:D]
    Wb = W_edge1[D:2 * D]
    Wc = W_edge1[2 * D:]
    bn2 = b_node.reshape(1, D)
    g2 = ln_g.reshape(1, D)
    bln2 = ln_b.reshape(1, D)
    be1_2 = b_edge1.reshape(1, D)
    bp1_2 = b_pos1.reshape(1, 64)
    bp2_2 = b_pos2.reshape(1, D)
    be2_2 = b_edge2.reshape(1, D)

    f32 = jnp.float32
    c_table = pl.pallas_call(
        functools.partial(_ctable_body, H),
        grid=(1,),
        in_specs=[
            pl.BlockSpec((2, 64), lambda i: (0, 0)),
            pl.BlockSpec((1, 64), lambda i: (0, 0)),
            pl.BlockSpec((64, D), lambda i: (0, 0)),
            pl.BlockSpec((1, D), lambda i: (0, 0)),
            pl.BlockSpec((D, D), lambda i: (0, 0)),
        ],
        out_specs=pl.BlockSpec((T, D), lambda i: (0, 0)),
        out_shape=jax.ShapeDtypeStruct((T, D), f32),
    )(W_pos1, bp1_2, W_pos2, bp2_2, Wc)

    nodes, adj, gidx, cidx, a_c, b_c = pl.pallas_call(
        functools.partial(_main_body, H),
        grid=(B,),
        in_specs=[
            pl.BlockSpec((1, C, N), lambda i: (i, 0, 0)),
            pl.BlockSpec((C, D), lambda i: (0, 0)),
            pl.BlockSpec((1, D), lambda i: (0, 0)),
            pl.BlockSpec((1, D), lambda i: (0, 0)),
            pl.BlockSpec((1, D), lambda i: (0, 0)),
            pl.BlockSpec((D, D), lambda i: (0, 0)),
            pl.BlockSpec((D, D), lambda i: (0, 0)),
            pl.BlockSpec((1, D), lambda i: (0, 0)),
        ],
        out_specs=[
            pl.BlockSpec((1, N, D), lambda i: (i, 0, 0)),
            pl.BlockSpec((1, N, K), lambda i: (i, 0, 0)),
            pl.BlockSpec((1, N, K), lambda i: (i, 0, 0)),
            pl.BlockSpec((1, N, K), lambda i: (i, 0, 0)),
            pl.BlockSpec((1, N, D), lambda i: (i, 0, 0)),
            pl.BlockSpec((1, N, D), lambda i: (i, 0, 0)),
        ],
        out_shape=[
            jax.ShapeDtypeStruct((B, N, D), f32),
            jax.ShapeDtypeStruct((B, N, K), jnp.int32),
            jax.ShapeDtypeStruct((B, N, K), jnp.int32),
            jax.ShapeDtypeStruct((B, N, K), jnp.int32),
            jax.ShapeDtypeStruct((B, N, D), f32),
            jax.ShapeDtypeStruct((B, N, D), f32),
        ],
    )(fm3, W_node, bn2, g2, bln2, Wa, Wb, be1_2)

    total = B * N * K
    n_chunks = total // (_NW * _CH)
    mesh = plsc.VectorSubcoreMesh(core_axis_name="c", subcore_axis_name="s")
    nb, cc = pl.kernel(
        functools.partial(_gather_body, n_chunks),
        mesh=mesh,
        out_type=[
            jax.ShapeDtypeStruct((total, D), f32),
            jax.ShapeDtypeStruct((total, D), f32),
        ],
        scratch_types=[
            pltpu.VMEM((_CH,), jnp.int32),
            pltpu.VMEM((_CH,), jnp.int32),
            pltpu.VMEM((_CH, D), f32),
            pltpu.VMEM((_CH, D), f32),
            pltpu.SemaphoreType.DMA,
            pltpu.SemaphoreType.DMA,
        ],
    )(b_c.reshape(B * N, D), c_table, gidx.reshape(total), cidx.reshape(total))

    edges = pl.pallas_call(
        _edge_body,
        grid=(B,),
        in_specs=[
            pl.BlockSpec((1, N, D), lambda i: (i, 0, 0)),
            pl.BlockSpec((1, N, K, D), lambda i: (i, 0, 0, 0)),
            pl.BlockSpec((1, N, K, D), lambda i: (i, 0, 0, 0)),
            pl.BlockSpec((D, D), lambda i: (0, 0)),
            pl.BlockSpec((1, D), lambda i: (0, 0)),
        ],
        out_specs=pl.BlockSpec((1, N, K, D), lambda i: (i, 0, 0, 0)),
        out_shape=jax.ShapeDtypeStruct((B, N, K, D), f32),
    )(a_c, nb.reshape(B, N, K, D), cc.reshape(B, N, K, D), W_edge2, be2_2)

    return (nodes, edges, adj)
